# baseline (speedup 1.0000x reference)
import functools

import numpy as np
import jax
import jax.numpy as jnp
from jax import lax
from jax.experimental import pallas as pl
from jax.experimental.pallas import tpu as pltpu
from jax.experimental.pallas import tpu_sc as plsc

_B, _L, _V = 4096, 20, 64
_P = 0.1
_DEL_SENTINEL = 127


def _np_rotl(x, r):
    return ((x << np.uint32(r)) | (x >> np.uint32(32 - r))).astype(np.uint32)


def _np_threefry2x32(k0, k1, x0, x1):
    rots = [13, 15, 26, 6, 17, 29, 16, 24]
    ks = [np.uint32(k0), np.uint32(k1),
          np.uint32(k0) ^ np.uint32(k1) ^ np.uint32(0x1BD11BDA)]
    x0 = (x0 + ks[0]).astype(np.uint32)
    x1 = (x1 + ks[1]).astype(np.uint32)
    for i in range(5):
        for r in (rots[0:4] if i % 2 == 0 else rots[4:8]):
            x0 = (x0 + x1).astype(np.uint32)
            x1 = _np_rotl(x1, r)
            x1 = x1 ^ x0
        x0 = (x0 + ks[(i + 1) % 3]).astype(np.uint32)
        x1 = (x1 + ks[(i + 2) % 3] + np.uint32(i + 1)).astype(np.uint32)
    return x0, x1


def _np_uniform(seed, shape):
    size = int(np.prod(shape))
    k0, k1 = np.uint32(seed >> 32), np.uint32(seed & 0xFFFFFFFF)
    idx = np.arange(size, dtype=np.uint64)
    x0 = (idx >> np.uint64(32)).astype(np.uint32)
    x1 = (idx & np.uint64(0xFFFFFFFF)).astype(np.uint32)
    y0, y1 = _np_threefry2x32(k0, k1, x0, x1)
    bits = y0 ^ y1
    f = ((bits >> np.uint32(9)) | np.uint32(0x3F800000)).view(np.float32)
    return np.maximum(np.float32(0.0), f - np.float32(1.0)).reshape(shape)


def _build_src():
    mask = _np_uniform(42, (_B, _L)) < np.float32(_P)
    src = np.full((_L, _B), _DEL_SENTINEL, np.int32)
    smax = 0
    for b in range(_B):
        kp = np.flatnonzero(~mask[b])
        src[: kp.size, b] = kp
        if kp.size:
            smax = max(smax, int((kp - np.arange(kp.size)).max()))
    return src, smax


_SRC, _S = _build_src()


_NC, _NS = 2, 16
_NW = _NC * _NS
_FLAT = _B * _L * _V
_FPW = _FLAT // _NW


_BCW = _B // _NW


def _sc_copy_body(m_hbm, p_hbm, cm_hbm, cp_hbm, sem0, sem1):
    wid = lax.axis_index("s") * _NC + lax.axis_index("c")
    col = pl.ds(wid * _BCW, _BCW)
    c0 = pltpu.async_copy(m_hbm.at[:, :, col], cm_hbm.at[:, :, col], sem0)
    c1 = pltpu.async_copy(p_hbm.at[:, :, col], cp_hbm.at[:, :, col], sem1)
    c0.wait()
    c1.wait()


@functools.cache
def _sc_copy():
    shp = jax.ShapeDtypeStruct((_L, _V, _B), jnp.float32)
    return pl.kernel(
        _sc_copy_body,
        out_type=(shp, shp),
        mesh=plsc.VectorSubcoreMesh(core_axis_name="c", subcore_axis_name="s",
                                    num_cores=_NC, num_subcores=_NS),
        scratch_types=[
            pltpu.SemaphoreType.DMA,
            pltpu.SemaphoreType.DMA,
        ],
        compiler_params=pltpu.CompilerParams(needs_layout_passes=False,
                                             use_tc_tiling_on_sc=True),
    )


def _tc_body(src_ref, m_ref, p_ref, nm_ref, np_ref):
    x = m_ref[...]
    p = p_ref[...]
    sv = src_ref[...]
    shift = sv - lax.broadcasted_iota(jnp.int32, sv.shape, 0)
    zpad = jnp.zeros((_S, _V) + x.shape[2:], jnp.float32)
    xp = jnp.concatenate([x, zpad], axis=0)
    acc = jnp.zeros_like(x)
    for s in range(_S + 1):
        w = (shift == s).astype(jnp.float32)[:, None, :]
        acc = acc + w * lax.slice_in_dim(xp, s, s + _L, axis=0)
    viota = lax.broadcasted_iota(jnp.int32, x.shape, 1)
    delw = (sv == _DEL_SENTINEL).astype(jnp.float32)[:, None, :]
    nm_ref[...] = acc + delw * (viota == 0).astype(jnp.float32)

    tot = jnp.sum(p, axis=1, keepdims=True)
    head = 1.0 - jnp.float32(1.0 - _P) * (tot - p[:, 0:1, :])
    np_ref[...] = jnp.where(viota == 0, head, jnp.float32(1.0 - _P) * p)


_BB = 256


@functools.cache
def _tc_call():
    grid = (_B // _BB,)
    blk = pl.BlockSpec((_L, _V, _BB), lambda i: (0, 0, i))
    sblk = pl.BlockSpec((_L, _BB), lambda i: (0, i))
    out = jax.ShapeDtypeStruct((_L, _V, _B), jnp.float32)
    return pl.pallas_call(
        _tc_body,
        grid=grid,
        in_specs=[sblk, blk, blk],
        out_specs=[blk, blk],
        out_shape=[out, out],
    )


def kernel(messages, probs):
    m_t = jnp.transpose(messages, (1, 2, 0))
    p_t = jnp.transpose(probs, (1, 2, 0))
    nm, npr = _tc_call()(jnp.asarray(_SRC), m_t, p_t)
    cm, cp = _sc_copy()(m_t, p_t)
    back = lambda a: jnp.transpose(a, (2, 0, 1))
    return back(nm), back(npr), back(cm), back(cp)
